# baseline (speedup 1.0000x reference)
import functools

import jax
import jax.numpy as jnp
from jax import lax
from jax.experimental import pallas as pl
from jax.experimental.pallas import tpu as pltpu
from jax.experimental.pallas import tpu_sc as plsc

N = 50000
E = 800000
C = 10
M = 32
G = 8
CG = C * G
GRP = 16
NG = CG // GRP

NP = 50176
BN = 1024
NBLK = NP // BN
TROWS = NP // 16

CH = 256
SUB = CH // 128
CHUNKS = 200
STEP = 8
GROUPS = CHUNKS // STEP
RRING = 4
IRING = 8
EPT = CH * CHUNKS
EP = 16 * EPT

_HIGH = jax.lax.Precision.DEFAULT


def _dot(a, b):
    return jax.lax.dot(a, b, precision=_HIGH)


@functools.lru_cache(maxsize=None)
def _sc_segsum(with_counts: bool):
    outs = [jax.ShapeDtypeStruct((NP, GRP), jnp.float32) for _ in range(NG)]
    if with_counts:
        outs.append(jax.ShapeDtypeStruct((NP, GRP), jnp.float32))

    mesh = plsc.VectorSubcoreMesh(core_axis_name="c", subcore_axis_name="s",
                                  num_cores=2, num_subcores=16)

    @functools.partial(
        pl.kernel, mesh=mesh, out_type=tuple(outs),
        compiler_params=pltpu.CompilerParams(use_tc_tiling_on_sc=False),
        scratch_types=[
            [pltpu.VMEM((2 * SUB, 128), jnp.int32)] * IRING,
            [pltpu.VMEM((CH, GRP), jnp.float32)] * RRING,
            pltpu.VMEM((128, GRP), jnp.float32),
            pltpu.VMEM_SHARED((NP, GRP), jnp.float32),
            [pltpu.SemaphoreType.DMA] * IRING,
            [pltpu.SemaphoreType.DMA] * RRING,
            [pltpu.SemaphoreType.DMA] * IRING,
        ],
    )
    def k(h0, h1, h2, h3, h4, edg, z2d, o1, *rest):
        houts = rest[:NG]
        outc = rest[NG] if with_counts else None
        scr = rest[NG + 1:] if with_counts else rest[NG:]
        idx, rows, ones, acc, isems, gsems, ssems = scr
        tabs = (h0, h1, h2, h3, h4)
        c = lax.axis_index("c")
        s = lax.axis_index("s")
        row0 = s * TROWS

        if with_counts:
            pltpu.sync_copy(o1, ones)

        def fire_idx(ci, r):
            pltpu.async_copy(edg.at[s, ci], idx[r], isems[r])

        def drain_idx(r):
            pltpu.make_async_copy(edg.at[0, 0], idx[r], isems[r]).wait()

        def drain_rows(sem):
            pltpu.make_async_copy(h0.at[pl.ds(0, CH)], rows[0], sem).wait()

        def sweep(tab, out):
            pltpu.sync_copy(z2d, acc.at[pl.ds(row0, TROWS)])
            plsc.subcore_barrier()

            def fire_gather(ci, r, ir):
                for j in range(SUB):
                    pltpu.async_copy(tab.at[idx[ir].at[j]],
                                     rows[r].at[pl.ds(j * 128, 128)],
                                     gsems[r])

            def fire_scatter(ci, r, ir):
                for j in range(SUB):
                    if tab is None:
                        pltpu.async_copy(ones, acc.at[idx[ir].at[SUB + j]],
                                         ssems[ir], add=True)
                    else:
                        pltpu.async_copy(rows[r].at[pl.ds(j * 128, 128)],
                                         acc.at[idx[ir].at[SUB + j]],
                                         ssems[ir], add=True)

            for r in range(IRING - 1):
                fire_idx(r, r)
            for r in range(RRING - 1):
                drain_idx(r)
                if tab is not None:
                    fire_gather(r, r, r)

            def group(g, _):
                for b in range(STEP):
                    ci = STEP * g + b
                    i_b = b
                    r_b = b % RRING
                    i_n = (b + IRING - 1) % IRING
                    r_n = (b + RRING - 1) % RRING

                    @pl.when(ci > 0)
                    def _():
                        drain_rows(ssems[i_n])

                    @pl.when(ci + IRING - 1 < CHUNKS)
                    def _():
                        fire_idx(ci + IRING - 1, i_n)

                    @pl.when(ci + RRING - 1 < CHUNKS)
                    def _():
                        drain_idx((b + RRING - 1) % IRING)
                        if tab is not None:
                            fire_gather(ci + RRING - 1, r_n,
                                        (b + RRING - 1) % IRING)

                    if tab is not None:
                        drain_rows(gsems[r_b])
                    fire_scatter(ci, r_b, i_b)
                return 0
            lax.fori_loop(0, GROUPS, group, 0)
            drain_rows(ssems[(CHUNKS - 1) % IRING])
            plsc.subcore_barrier()

            pltpu.sync_copy(acc.at[pl.ds(row0, TROWS)],
                            out.at[pl.ds(row0, TROWS)])

        @pl.when(c == 0)
        def _():
            sweep(tabs[0], houts[0])
            sweep(tabs[1], houts[1])
            sweep(tabs[2], houts[2])

        @pl.when(c == 1)
        def _():
            sweep(tabs[3], houts[3])
            sweep(tabs[4], houts[4])
            if with_counts:
                sweep(None, outc)

    return k


def _onehot(x_blk):
    iota = lax.broadcasted_iota(jnp.int32, (BN, M), 1).astype(jnp.float32)
    return (x_blk == iota).astype(jnp.float32)


def _layer0_body(x_ref, t0_ref, sj_ref, bc_ref, ll_ref, *h_refs):
    oh = _onehot(x_ref[...])
    u = _dot(oh, t0_ref[...])
    z = _dot(u, sj_ref[...])
    ll_ref[...] = jnp.log(z)
    h = u / _dot(z, bc_ref[...])
    for q in range(NG):
        h_refs[q][...] = h[:, q * GRP:(q + 1) * GRP]


def _layerL_body(x_ref, s0_ref, s1_ref, s2_ref, s3_ref, s4_ref, cnt_ref,
                 st_ref, wq_ref, tb_ref, sj_ref, bc_ref,
                 ll_ref, *h_refs):
    inv = 1.0 / jnp.maximum(cnt_ref[...][:, :1], 1.0)
    aggr = jnp.concatenate(
        [s0_ref[...], s1_ref[...], s2_ref[...], s3_ref[...], s4_ref[...]],
        axis=1) * inv
    oh = _onehot(x_ref[...])
    sx = _dot(oh, st_ref[...])
    z = _dot(sx * aggr, sj_ref[...])
    ll_ref[...] = jnp.log(z)
    r = _dot(aggr, wq_ref[...])
    h = _dot(oh, tb_ref[...]) * r / _dot(z, bc_ref[...])
    for q in range(NG):
        h_refs[q][...] = h[:, q * GRP:(q + 1) * GRP]


def _nspec(w):
    return pl.BlockSpec((BN, w), lambda i: (i, 0))


def _wspec(shape):
    return pl.BlockSpec(shape, lambda i: (0, 0))


_H_OUT = ([jax.ShapeDtypeStruct((NP, G), jnp.float32)] +
          [jax.ShapeDtypeStruct((NP, GRP), jnp.float32)] * NG)
_H_OUT_SPECS = [_nspec(G)] + [_nspec(GRP)] * NG


def _dense0(xf, t0, sj, bc):
    return pl.pallas_call(
        _layer0_body,
        grid=(NBLK,),
        in_specs=[_nspec(1), _wspec((M, CG)), _wspec((CG, G)),
                  _wspec((G, CG))],
        out_specs=_H_OUT_SPECS,
        out_shape=_H_OUT,
    )(xf, t0, sj, bc)


def _denseL(xf, sums, cnt, st, wq, tb, sj, bc):
    return pl.pallas_call(
        _layerL_body,
        grid=(NBLK,),
        in_specs=([_nspec(1)] + [_nspec(GRP)] * NG +
                  [_nspec(GRP), _wspec((M, CG)), _wspec((CG, CG)),
                   _wspec((M, CG)), _wspec((CG, G)), _wspec((G, CG))]),
        out_specs=_H_OUT_SPECS,
        out_shape=_H_OUT,
    )(xf, *sums, cnt, st, wq, tb, sj, bc)


def kernel(x, edge_index, B0, Pi0, Q_layers, B_layers):
    B0s = jax.nn.softmax(B0, axis=1)
    Pi = jax.nn.softmax(Pi0, axis=0)
    t0 = jnp.transpose(B0s * Pi[:, None, :], (1, 0, 2)).reshape(M, CG)
    sj = jnp.tile(jnp.eye(G, dtype=jnp.float32), (C, 1))
    bc = sj.T

    def layer_tables(lq, lb):
        Q = jax.nn.softmax(lq, axis=0)
        B = jax.nn.softmax(lb, axis=1)
        st = jnp.einsum("img,ijg->mjg", B, Q).reshape(M, CG)
        wq = jnp.einsum("ijg,gh->jgih", Q,
                        jnp.eye(G, dtype=jnp.float32)).reshape(CG, CG)
        tb = jnp.transpose(B, (1, 0, 2)).reshape(M, CG)
        return st, wq, tb

    st1, wq1, tb1 = layer_tables(Q_layers[0], B_layers[0])
    st2, wq2, tb2 = layer_tables(Q_layers[1], B_layers[1])

    xf = jnp.pad(x.astype(jnp.float32), (0, NP - N)).reshape(NP, 1)
    src = jnp.asarray(edge_index[1], jnp.int32)
    dst = jnp.asarray(edge_index[0], jnp.int32)
    srcp = jnp.pad(src, (0, EP - E)).reshape(16, CHUNKS, SUB, 128)
    dstp = jnp.pad(dst, (0, EP - E),
                   constant_values=NP - 1).reshape(16, CHUNKS, SUB, 128)
    edg = jnp.concatenate([srcp, dstp], axis=2)
    z2d = jnp.zeros((TROWS, GRP), jnp.float32)
    o1 = jnp.ones((128, GRP), jnp.float32)

    ll0, *h0 = _dense0(xf, t0, sj, bc)

    *s0, cnt16 = _sc_segsum(True)(*h0, edg, z2d, o1)
    ll1, *h1 = _denseL(xf, s0, cnt16, st1, wq1, tb1, sj, bc)

    s1 = _sc_segsum(False)(*h1, edg, z2d, o1)
    ll2, *_ = _denseL(xf, list(s1), cnt16, st2, wq2, tb2, sj, bc)

    return jnp.stack([ll0[:N], ll1[:N], ll2[:N]], axis=1)


# speedup vs baseline: 1.4000x; 1.4000x over previous
import functools

import jax
import jax.numpy as jnp
from jax import lax
from jax.experimental import pallas as pl
from jax.experimental.pallas import tpu as pltpu
from jax.experimental.pallas import tpu_sc as plsc

N = 50000
E = 800000
C = 10
M = 32
G = 8
CG = C * G
GRP = 16
NG = CG // GRP

NP = 50176
BN = 1024
NBLK = NP // BN
TROWS = NP // 16

CH = 1792
SUB = CH // 128
CHUNKS = 28
PAIRS = CHUNKS // 2
EPT = CH * CHUNKS
EP = 16 * EPT

_HIGH = jax.lax.Precision.DEFAULT


def _dot(a, b):
    return jax.lax.dot(a, b, precision=_HIGH)


@functools.lru_cache(maxsize=None)
def _sc_segsum(with_counts: bool):
    outs = [jax.ShapeDtypeStruct((NP, GRP), jnp.float32) for _ in range(NG)]
    if with_counts:
        outs.append(jax.ShapeDtypeStruct((NP, GRP), jnp.float32))

    mesh = plsc.VectorSubcoreMesh(core_axis_name="c", subcore_axis_name="s",
                                  num_cores=2, num_subcores=16)

    @functools.partial(
        pl.kernel, mesh=mesh, out_type=tuple(outs),
        compiler_params=pltpu.CompilerParams(use_tc_tiling_on_sc=False),
        scratch_types=[
            [pltpu.VMEM((2 * SUB, 128), jnp.int32)] * 2,
            [pltpu.VMEM((CH, GRP), jnp.float32)] * 2,
            pltpu.VMEM((128, GRP), jnp.float32),
            pltpu.VMEM_SHARED((NP, GRP), jnp.float32),
            [pltpu.SemaphoreType.DMA] * 2,
            pltpu.SemaphoreType.DMA,
            pltpu.SemaphoreType.DMA,
        ],
    )
    def k(h0, h1, h2, h3, h4, edg, z2d, o1, *rest):
        houts = rest[:NG]
        outc = rest[NG] if with_counts else None
        scr = rest[NG + 1:] if with_counts else rest[NG:]
        idx, rows, ones, acc, gsems, ssem, isem = scr
        tabs = (h0, h1, h2, h3, h4)
        c = lax.axis_index("c")
        s = lax.axis_index("s")
        row0 = s * TROWS

        if with_counts:
            pltpu.sync_copy(o1, ones)

        def fire_idx(ci, b):
            pltpu.async_copy(edg.at[s, ci], idx[b], isem)

        def drain_idx(b):
            pltpu.make_async_copy(edg.at[0, 0], idx[b], isem).wait()

        def drain_rows(sem):
            pltpu.make_async_copy(h0.at[pl.ds(0, CH)], rows[0], sem).wait()

        def sweep(tab, out):
            pltpu.sync_copy(z2d, acc.at[pl.ds(row0, TROWS)])
            plsc.subcore_barrier()

            def fire_gathers(b):
                for j in range(SUB):
                    pltpu.async_copy(tab.at[idx[b].at[j]],
                                     rows[b].at[pl.ds(j * 128, 128)],
                                     gsems[b])

            def fire_scatters(b):
                for j in range(SUB):
                    if tab is None:
                        pltpu.async_copy(ones, acc.at[idx[b].at[SUB + j]],
                                         ssem, add=True)
                    else:
                        pltpu.async_copy(rows[b].at[pl.ds(j * 128, 128)],
                                         acc.at[idx[b].at[SUB + j]],
                                         ssem, add=True)

            fire_idx(0, 0)
            drain_idx(0)
            if tab is not None:
                fire_gathers(0)

            def pair(p, _):
                for b in (0, 1):
                    ci = 2 * p + b
                    nb = 1 - b

                    @pl.when(ci > 0)
                    def _():
                        drain_rows(ssem)

                    @pl.when(ci + 1 < CHUNKS)
                    def _():
                        fire_idx(ci + 1, nb)
                        drain_idx(nb)
                        if tab is not None:
                            fire_gathers(nb)

                    if tab is not None:
                        drain_rows(gsems[b])
                    fire_scatters(b)
                return 0
            lax.fori_loop(0, PAIRS, pair, 0)
            drain_rows(ssem)
            plsc.subcore_barrier()

            pltpu.sync_copy(acc.at[pl.ds(row0, TROWS)],
                            out.at[pl.ds(row0, TROWS)])

        @pl.when(c == 0)
        def _():
            sweep(tabs[0], houts[0])
            sweep(tabs[1], houts[1])
            sweep(tabs[2], houts[2])

        @pl.when(c == 1)
        def _():
            sweep(tabs[3], houts[3])
            sweep(tabs[4], houts[4])
            if with_counts:
                sweep(None, outc)

    return k


def _onehot(x_blk):
    iota = lax.broadcasted_iota(jnp.int32, (BN, M), 1).astype(jnp.float32)
    return (x_blk == iota).astype(jnp.float32)


def _layer0_body(x_ref, t0_ref, sj_ref, bc_ref, ll_ref, *h_refs):
    oh = _onehot(x_ref[...])
    u = _dot(oh, t0_ref[...])
    z = _dot(u, sj_ref[...])
    ll_ref[...] = jnp.log(z)
    h = u / _dot(z, bc_ref[...])
    for q in range(NG):
        h_refs[q][...] = h[:, q * GRP:(q + 1) * GRP]


def _layerL_body(x_ref, s0_ref, s1_ref, s2_ref, s3_ref, s4_ref, cnt_ref,
                 st_ref, wq_ref, tb_ref, sj_ref, bc_ref,
                 ll_ref, *h_refs):
    inv = 1.0 / jnp.maximum(cnt_ref[...][:, :1], 1.0)
    aggr = jnp.concatenate(
        [s0_ref[...], s1_ref[...], s2_ref[...], s3_ref[...], s4_ref[...]],
        axis=1) * inv
    oh = _onehot(x_ref[...])
    sx = _dot(oh, st_ref[...])
    z = _dot(sx * aggr, sj_ref[...])
    ll_ref[...] = jnp.log(z)
    r = _dot(aggr, wq_ref[...])
    h = _dot(oh, tb_ref[...]) * r / _dot(z, bc_ref[...])
    for q in range(NG):
        h_refs[q][...] = h[:, q * GRP:(q + 1) * GRP]


def _nspec(w):
    return pl.BlockSpec((BN, w), lambda i: (i, 0))


def _wspec(shape):
    return pl.BlockSpec(shape, lambda i: (0, 0))


_H_OUT = ([jax.ShapeDtypeStruct((NP, G), jnp.float32)] +
          [jax.ShapeDtypeStruct((NP, GRP), jnp.float32)] * NG)
_H_OUT_SPECS = [_nspec(G)] + [_nspec(GRP)] * NG


def _dense0(xf, t0, sj, bc):
    return pl.pallas_call(
        _layer0_body,
        grid=(NBLK,),
        in_specs=[_nspec(1), _wspec((M, CG)), _wspec((CG, G)),
                  _wspec((G, CG))],
        out_specs=_H_OUT_SPECS,
        out_shape=_H_OUT,
    )(xf, t0, sj, bc)


def _denseL(xf, sums, cnt, st, wq, tb, sj, bc):
    return pl.pallas_call(
        _layerL_body,
        grid=(NBLK,),
        in_specs=([_nspec(1)] + [_nspec(GRP)] * NG +
                  [_nspec(GRP), _wspec((M, CG)), _wspec((CG, CG)),
                   _wspec((M, CG)), _wspec((CG, G)), _wspec((G, CG))]),
        out_specs=_H_OUT_SPECS,
        out_shape=_H_OUT,
    )(xf, *sums, cnt, st, wq, tb, sj, bc)


def kernel(x, edge_index, B0, Pi0, Q_layers, B_layers):
    B0s = jax.nn.softmax(B0, axis=1)
    Pi = jax.nn.softmax(Pi0, axis=0)
    t0 = jnp.transpose(B0s * Pi[:, None, :], (1, 0, 2)).reshape(M, CG)
    sj = jnp.tile(jnp.eye(G, dtype=jnp.float32), (C, 1))
    bc = sj.T

    def layer_tables(lq, lb):
        Q = jax.nn.softmax(lq, axis=0)
        B = jax.nn.softmax(lb, axis=1)
        st = jnp.einsum("img,ijg->mjg", B, Q).reshape(M, CG)
        wq = jnp.einsum("ijg,gh->jgih", Q,
                        jnp.eye(G, dtype=jnp.float32)).reshape(CG, CG)
        tb = jnp.transpose(B, (1, 0, 2)).reshape(M, CG)
        return st, wq, tb

    st1, wq1, tb1 = layer_tables(Q_layers[0], B_layers[0])
    st2, wq2, tb2 = layer_tables(Q_layers[1], B_layers[1])

    xf = jnp.pad(x.astype(jnp.float32), (0, NP - N)).reshape(NP, 1)
    src = jnp.asarray(edge_index[1], jnp.int32)
    dst = jnp.asarray(edge_index[0], jnp.int32)
    srcp = jnp.pad(src, (0, EP - E)).reshape(16, CHUNKS, SUB, 128)
    dstp = jnp.pad(dst, (0, EP - E),
                   constant_values=NP - 1).reshape(16, CHUNKS, SUB, 128)
    edg = jnp.concatenate([srcp, dstp], axis=2)
    z2d = jnp.zeros((TROWS, GRP), jnp.float32)
    o1 = jnp.ones((128, GRP), jnp.float32)

    ll0, *h0 = _dense0(xf, t0, sj, bc)

    *s0, cnt16 = _sc_segsum(True)(*h0, edg, z2d, o1)
    ll1, *h1 = _denseL(xf, s0, cnt16, st1, wq1, tb1, sj, bc)

    s1 = _sc_segsum(False)(*h1, edg, z2d, o1)
    ll2, *_ = _denseL(xf, list(s1), cnt16, st2, wq2, tb2, sj, bc)

    return jnp.stack([ll0[:N], ll1[:N], ll2[:N]], axis=1)
